# Initial kernel scaffold; baseline (speedup 1.0000x reference)
import jax
import jax.numpy as jnp
from jax.experimental import pallas as pl


def kernel(image_input, W1, b1, W2, b2, Wfc, bfc, LUT0, LUT1, LUT2):
    raise NotImplementedError("write your pallas kernel here")


# speedup vs baseline: 428.5801x; 428.5801x over previous
import functools

import jax
import jax.numpy as jnp
from jax import lax
from jax.experimental import pallas as pl
from jax.experimental.pallas import tpu as pltpu
from jax.experimental.pallas import tpu_sc as plsc

_DIM = 33
_D2 = _DIM * _DIM
_D3 = _DIM * _D2
_H, _W = 1080, 1920
_NPIX = _H * _W
_NWORKER = 32
_PPW = _NPIX // _NWORKER
_CHUNK = 1440
_NCHUNK = _PPW // _CHUNK
_GROUPS = _CHUNK // 16
_LUT_PAD = 108544

_mesh = plsc.VectorSubcoreMesh(core_axis_name="c", subcore_axis_name="s")


@functools.partial(
    pl.kernel,
    mesh=_mesh,
    compiler_params=pltpu.CompilerParams(needs_layout_passes=False),
    out_type=jax.ShapeDtypeStruct((3 * _NPIX,), jnp.float32),
    scratch_types=[
        pltpu.VMEM((_LUT_PAD,), jnp.float32),
        pltpu.VMEM((_CHUNK,), jnp.float32),
        pltpu.VMEM((_CHUNK,), jnp.float32),
        pltpu.VMEM((_CHUNK,), jnp.float32),
        pltpu.VMEM((_CHUNK,), jnp.float32),
        pltpu.VMEM((_CHUNK,), jnp.float32),
        pltpu.VMEM((_CHUNK,), jnp.float32),
    ],
)
def _trilerp(lut_hbm, img_hbm, out_hbm, lut_v,
             in_r, in_g, in_b, o_r, o_g, o_b):
    in_v = (in_r, in_g, in_b)
    out_v = (o_r, o_g, o_b)
    wid = lax.axis_index("s") * 2 + lax.axis_index("c")
    base = wid * _PPW
    pltpu.sync_copy(lut_hbm, lut_v)

    def chunk_body(j, carry):
        off = base + j * _CHUNK
        for c in range(3):
            pltpu.sync_copy(img_hbm.at[pl.ds(c * _NPIX + off, _CHUNK)],
                            in_v[c])

        def grp(i, carry2):
            s = i * 16
            r = in_v[0][pl.ds(s, 16)] * float(_DIM - 1)
            g = in_v[1][pl.ds(s, 16)] * float(_DIM - 1)
            b = in_v[2][pl.ds(s, 16)] * float(_DIM - 1)
            r0 = jnp.clip(r.astype(jnp.int32), 0, _DIM - 2)
            g0 = jnp.clip(g.astype(jnp.int32), 0, _DIM - 2)
            b0 = jnp.clip(b.astype(jnp.int32), 0, _DIM - 2)
            fr = r - r0.astype(jnp.float32)
            fg = g - g0.astype(jnp.float32)
            fb = b - b0.astype(jnp.float32)
            idx = r0 * _D2 + g0 * _DIM + b0
            for c in range(3):
                p = idx + c * _D3
                v000 = plsc.load_gather(lut_v, [p])
                v001 = plsc.load_gather(lut_v, [p + 1])
                v010 = plsc.load_gather(lut_v, [p + _DIM])
                v011 = plsc.load_gather(lut_v, [p + (_DIM + 1)])
                v100 = plsc.load_gather(lut_v, [p + _D2])
                v101 = plsc.load_gather(lut_v, [p + (_D2 + 1)])
                v110 = plsc.load_gather(lut_v, [p + (_D2 + _DIM)])
                v111 = plsc.load_gather(lut_v, [p + (_D2 + _DIM + 1)])
                c00 = v000 + fb * (v001 - v000)
                c01 = v010 + fb * (v011 - v010)
                c10 = v100 + fb * (v101 - v100)
                c11 = v110 + fb * (v111 - v110)
                c0 = c00 + fg * (c01 - c00)
                c1 = c10 + fg * (c11 - c10)
                out_v[c][pl.ds(s, 16)] = c0 + fr * (c1 - c0)
            return carry2

        lax.fori_loop(0, _GROUPS, grp, 0)
        for c in range(3):
            pltpu.sync_copy(out_v[c],
                            out_hbm.at[pl.ds(c * _NPIX + off, _CHUNK)])
        return carry

    lax.fori_loop(0, _NCHUNK, chunk_body, 0)


def _conv(x, w, b, stride):
    y = lax.conv_general_dilated(
        x, w, (stride, stride), "SAME",
        dimension_numbers=("NCHW", "OIHW", "NCHW"))
    return y + b[None, :, None, None]


def _lrelu(x):
    return jnp.where(x > 0, x, 0.2 * x)


def _classifier(x, W1, b1, W2, b2, Wfc, bfc):
    x = jax.image.resize(x, (x.shape[0], x.shape[1], 256, 256),
                         method="bilinear")
    h = _lrelu(_conv(x, W1, b1, 2))
    h = _lrelu(_conv(h, W2, b2, 2))
    h = jnp.mean(h, axis=(2, 3))
    return h @ Wfc.T + bfc


def kernel(image_input, W1, b1, W2, b2, Wfc, bfc, LUT0, LUT1, LUT2):
    pred = jnp.squeeze(_classifier(image_input, W1, b1, W2, b2, Wfc, bfc))
    flut = pred[0] * LUT0 + pred[1] * LUT1 + pred[2] * LUT2
    lut_flat = jnp.pad(flut.reshape(-1), (0, _LUT_PAD - 3 * _D3))
    img = image_input.reshape(3 * _NPIX)
    out = _trilerp(lut_flat, img)
    return out.reshape(1, 3, _H, _W)


# speedup vs baseline: 509.9663x; 1.1899x over previous
import functools

import jax
import jax.numpy as jnp
from jax import lax
from jax.experimental import pallas as pl
from jax.experimental.pallas import tpu as pltpu
from jax.experimental.pallas import tpu_sc as plsc

_DIM = 33
_D2 = _DIM * _DIM
_D3 = _DIM * _D2
_H, _W = 1080, 1920
_NPIX = _H * _W
_NWORKER = 32
_PPW = _NPIX // _NWORKER
_CHUNK = 1200
_NCHUNK = _PPW // _CHUNK
_GROUPS = _CHUNK // 16
_LUT_PAD = 108544

_mesh = plsc.VectorSubcoreMesh(core_axis_name="c", subcore_axis_name="s")


@functools.partial(
    pl.kernel,
    mesh=_mesh,
    compiler_params=pltpu.CompilerParams(needs_layout_passes=False),
    out_type=jax.ShapeDtypeStruct((3 * _NPIX,), jnp.float32),
    scratch_types=[
        pltpu.VMEM((_LUT_PAD,), jnp.float32),
    ] + [pltpu.VMEM((_CHUNK,), jnp.float32) for _ in range(12)] + [
        pltpu.SemaphoreType.DMA,
        pltpu.SemaphoreType.DMA,
        pltpu.SemaphoreType.DMA,
        pltpu.SemaphoreType.DMA,
    ],
)
def _trilerp(lut_hbm, img_hbm, out_hbm, lut_v,
             ir0, ig0, ib0, ir1, ig1, ib1,
             or0, og0, ob0, or1, og1, ob1,
             sin0, sin1, sout0, sout1):
    in_bufs = ((ir0, ig0, ib0), (ir1, ig1, ib1))
    out_bufs = ((or0, og0, ob0), (or1, og1, ob1))
    sem_in = (sin0, sin1)
    sem_out = (sout0, sout1)
    wid = lax.axis_index("s") * 2 + lax.axis_index("c")
    base = wid * _PPW
    pltpu.sync_copy(lut_hbm, lut_v)

    def start_in(j, b):
        off = base + j * _CHUNK
        for c in range(3):
            pltpu.async_copy(img_hbm.at[pl.ds(c * _NPIX + off, _CHUNK)],
                             in_bufs[b][c], sem_in[b])

    def wait_in(b):
        for c in range(3):
            pltpu.make_async_copy(img_hbm.at[pl.ds(0, _CHUNK)],
                                  in_bufs[b][c], sem_in[b]).wait()

    def start_out(j, b):
        off = base + j * _CHUNK
        for c in range(3):
            pltpu.async_copy(out_bufs[b][c],
                             out_hbm.at[pl.ds(c * _NPIX + off, _CHUNK)],
                             sem_out[b])

    def wait_out(b):
        for c in range(3):
            pltpu.make_async_copy(
                out_bufs[b][c],
                out_hbm.at[pl.ds(0, _CHUNK)], sem_out[b]).wait()

    def compute_chunk(in_v, out_v):
        def grp(i, carry2):
            s = i * 16
            r = in_v[0][pl.ds(s, 16)] * float(_DIM - 1)
            g = in_v[1][pl.ds(s, 16)] * float(_DIM - 1)
            b = in_v[2][pl.ds(s, 16)] * float(_DIM - 1)
            r0 = jnp.clip(r.astype(jnp.int32), 0, _DIM - 2)
            g0 = jnp.clip(g.astype(jnp.int32), 0, _DIM - 2)
            b0 = jnp.clip(b.astype(jnp.int32), 0, _DIM - 2)
            fr = r - r0.astype(jnp.float32)
            fg = g - g0.astype(jnp.float32)
            fb = b - b0.astype(jnp.float32)
            idx = r0 * _D2 + g0 * _DIM + b0
            for c in range(3):
                p = idx + c * _D3
                v000 = plsc.load_gather(lut_v, [p])
                v001 = plsc.load_gather(lut_v, [p + 1])
                v010 = plsc.load_gather(lut_v, [p + _DIM])
                v011 = plsc.load_gather(lut_v, [p + (_DIM + 1)])
                v100 = plsc.load_gather(lut_v, [p + _D2])
                v101 = plsc.load_gather(lut_v, [p + (_D2 + 1)])
                v110 = plsc.load_gather(lut_v, [p + (_D2 + _DIM)])
                v111 = plsc.load_gather(lut_v, [p + (_D2 + _DIM + 1)])
                c00 = v000 + fb * (v001 - v000)
                c01 = v010 + fb * (v011 - v010)
                c10 = v100 + fb * (v101 - v100)
                c11 = v110 + fb * (v111 - v110)
                c0 = c00 + fg * (c01 - c00)
                c1 = c10 + fg * (c11 - c10)
                out_v[c][pl.ds(s, 16)] = c0 + fr * (c1 - c0)
            return carry2

        lax.fori_loop(0, _GROUPS, grp, 0)

    start_in(0, 0)
    start_in(1, 1)

    def chunk_body(jj, carry):
        for b in range(2):
            j = jj * 2 + b
            wait_in(b)

            @pl.when(jj >= 1)
            def _():
                wait_out(b)

            compute_chunk(in_bufs[b], out_bufs[b])
            start_out(j, b)

            @pl.when(jj < _NCHUNK // 2 - 1)
            def _():
                start_in(j + 2, b)
        return carry

    lax.fori_loop(0, _NCHUNK // 2, chunk_body, 0)
    wait_out(0)
    wait_out(1)


def _conv(x, w, b, stride):
    y = lax.conv_general_dilated(
        x, w, (stride, stride), "SAME",
        dimension_numbers=("NCHW", "OIHW", "NCHW"))
    return y + b[None, :, None, None]


def _lrelu(x):
    return jnp.where(x > 0, x, 0.2 * x)


def _classifier(x, W1, b1, W2, b2, Wfc, bfc):
    x = jax.image.resize(x, (x.shape[0], x.shape[1], 256, 256),
                         method="bilinear")
    h = _lrelu(_conv(x, W1, b1, 2))
    h = _lrelu(_conv(h, W2, b2, 2))
    h = jnp.mean(h, axis=(2, 3))
    return h @ Wfc.T + bfc


def kernel(image_input, W1, b1, W2, b2, Wfc, bfc, LUT0, LUT1, LUT2):
    pred = jnp.squeeze(_classifier(image_input, W1, b1, W2, b2, Wfc, bfc))
    flut = pred[0] * LUT0 + pred[1] * LUT1 + pred[2] * LUT2
    lut_flat = jnp.pad(flut.reshape(-1), (0, _LUT_PAD - 3 * _D3))
    img = image_input.reshape(3 * _NPIX)
    out = _trilerp(lut_flat, img)
    return out.reshape(1, 3, _H, _W)


# speedup vs baseline: 672.7271x; 1.3192x over previous
import functools

import jax
import jax.numpy as jnp
from jax import lax
from jax.experimental import pallas as pl
from jax.experimental.pallas import tpu as pltpu
from jax.experimental.pallas import tpu_sc as plsc

_DIM = 33
_D2 = _DIM * _DIM
_D3 = _DIM * _D2
_H, _W = 1080, 1920
_NPIX = _H * _W
_NWORKER = 32
_PPW = _NPIX // _NWORKER
_CHUNK = 1200
_NCHUNK = _PPW // _CHUNK
_GROUPS = _CHUNK // 16
_LUT_PAD = 108544

@functools.lru_cache(maxsize=1)
def _build_trilerp():
    mesh = plsc.VectorSubcoreMesh(core_axis_name="c", subcore_axis_name="s")
    return functools.partial(
        pl.kernel,
        mesh=mesh,
        compiler_params=pltpu.CompilerParams(needs_layout_passes=False),
        out_type=jax.ShapeDtypeStruct((3 * _NPIX,), jnp.float32),
        scratch_types=[
            pltpu.VMEM((_LUT_PAD,), jnp.float32),
        ] + [pltpu.VMEM((_CHUNK,), jnp.float32) for _ in range(12)] + [
            pltpu.SemaphoreType.DMA,
            pltpu.SemaphoreType.DMA,
            pltpu.SemaphoreType.DMA,
            pltpu.SemaphoreType.DMA,
        ],
    )(_trilerp_body)


def _trilerp_body(lut_hbm, img_hbm, out_hbm, lut_v,
                  ir0, ig0, ib0, ir1, ig1, ib1,
                  or0, og0, ob0, or1, og1, ob1,
                  sin0, sin1, sout0, sout1):
    in_bufs = ((ir0, ig0, ib0), (ir1, ig1, ib1))
    out_bufs = ((or0, og0, ob0), (or1, og1, ob1))
    sem_in = (sin0, sin1)
    sem_out = (sout0, sout1)
    wid = lax.axis_index("s") * 2 + lax.axis_index("c")
    base = wid * _PPW
    pltpu.sync_copy(lut_hbm, lut_v)

    def start_in(j, b):
        off = base + j * _CHUNK
        for c in range(3):
            pltpu.async_copy(img_hbm.at[pl.ds(c * _NPIX + off, _CHUNK)],
                             in_bufs[b][c], sem_in[b])

    def wait_in(b):
        for c in range(3):
            pltpu.make_async_copy(img_hbm.at[pl.ds(0, _CHUNK)],
                                  in_bufs[b][c], sem_in[b]).wait()

    def start_out(j, b):
        off = base + j * _CHUNK
        for c in range(3):
            pltpu.async_copy(out_bufs[b][c],
                             out_hbm.at[pl.ds(c * _NPIX + off, _CHUNK)],
                             sem_out[b])

    def wait_out(b):
        for c in range(3):
            pltpu.make_async_copy(
                out_bufs[b][c],
                out_hbm.at[pl.ds(0, _CHUNK)], sem_out[b]).wait()

    def compute_chunk(in_v, out_v):
        def grp(i, carry2):
            s = i * 16
            r = in_v[0][pl.ds(s, 16)] * float(_DIM - 1)
            g = in_v[1][pl.ds(s, 16)] * float(_DIM - 1)
            b = in_v[2][pl.ds(s, 16)] * float(_DIM - 1)
            r0 = jnp.clip(r.astype(jnp.int32), 0, _DIM - 2)
            g0 = jnp.clip(g.astype(jnp.int32), 0, _DIM - 2)
            b0 = jnp.clip(b.astype(jnp.int32), 0, _DIM - 2)
            fr = r - r0.astype(jnp.float32)
            fg = g - g0.astype(jnp.float32)
            fb = b - b0.astype(jnp.float32)
            idx = r0 * _D2 + g0 * _DIM + b0
            for c in range(3):
                p = idx + c * _D3
                v000 = plsc.load_gather(lut_v, [p])
                v001 = plsc.load_gather(lut_v, [p + 1])
                v010 = plsc.load_gather(lut_v, [p + _DIM])
                v011 = plsc.load_gather(lut_v, [p + (_DIM + 1)])
                v100 = plsc.load_gather(lut_v, [p + _D2])
                v101 = plsc.load_gather(lut_v, [p + (_D2 + 1)])
                v110 = plsc.load_gather(lut_v, [p + (_D2 + _DIM)])
                v111 = plsc.load_gather(lut_v, [p + (_D2 + _DIM + 1)])
                c00 = v000 + fb * (v001 - v000)
                c01 = v010 + fb * (v011 - v010)
                c10 = v100 + fb * (v101 - v100)
                c11 = v110 + fb * (v111 - v110)
                c0 = c00 + fg * (c01 - c00)
                c1 = c10 + fg * (c11 - c10)
                out_v[c][pl.ds(s, 16)] = c0 + fr * (c1 - c0)
            return carry2

        lax.fori_loop(0, _GROUPS, grp, 0)

    start_in(0, 0)
    start_in(1, 1)

    def chunk_body(jj, carry):
        for b in range(2):
            j = jj * 2 + b
            wait_in(b)

            @pl.when(jj >= 1)
            def _():
                wait_out(b)

            compute_chunk(in_bufs[b], out_bufs[b])
            start_out(j, b)

            @pl.when(jj < _NCHUNK // 2 - 1)
            def _():
                start_in(j + 2, b)
        return carry

    lax.fori_loop(0, _NCHUNK // 2, chunk_body, 0)
    wait_out(0)
    wait_out(1)


@functools.lru_cache(maxsize=1)
def _consts():
    import numpy as np

    def wmat(in_size, out_size):
        inv_scale = np.float32(1.0) / np.float32(out_size / in_size)
        kernel_scale = max(float(inv_scale), 1.0)
        sample_f = ((np.arange(out_size, dtype=np.float32) + 0.5) * inv_scale
                    - 0.5)
        x = np.abs(sample_f[None, :]
                   - np.arange(in_size, dtype=np.float32)[:, None])
        x = x / np.float32(kernel_scale)
        w = np.maximum(np.float32(0), np.float32(1) - x).astype(np.float32)
        total = w.sum(axis=0, keepdims=True, dtype=np.float32)
        w = np.where(np.abs(total) > 1000.0 * np.finfo(np.float32).eps,
                     w / np.where(total != 0, total, 1), 0).astype(np.float32)
        valid = (sample_f >= -0.5) & (sample_f <= in_size - 0.5)
        return np.where(valid[None, :], w, 0).astype(np.float32)

    Ah = np.ascontiguousarray(wmat(_H, 256).T)
    AwT = wmat(_W, 256)
    sxT = []
    for dx in range(3):
        m = np.zeros((256, 128), np.float32)
        for x in range(128):
            j = 2 * x + dx
            if j < 256:
                m[j, x] = 1.0
        sxT.append(m)
    s2xT = []
    for dx in range(3):
        m = np.zeros((128, 64), np.float32)
        for x in range(64):
            j = 2 * x + dx
            if j < 128:
                m[j, x] = 1.0
        s2xT.append(m)
    m1 = np.ones((384, 1), np.float32)
    for c in range(3):
        m1[c * 128 + 127, 0] = 0.0
    m2 = np.ones((1024, 1), np.float32)
    for c in range(16):
        m2[c * 64 + 63, 0] = 0.0
    return (Ah, AwT, sxT[0], sxT[1], sxT[2], s2xT[0], s2xT[1], s2xT[2],
            m1, m2)


def _lrelu(x):
    return jnp.where(x > 0, x, 0.2 * x)


def _rowsel(M, dy, mask):
    n, L = M.shape
    M3 = M.reshape(n // 2, 2, L)
    even = M3[:, 0, :].reshape(n // 2, L)
    if dy == 0:
        return even
    if dy == 1:
        return M3[:, 1, :].reshape(n // 2, L)
    sh = jnp.concatenate([even[1:], jnp.zeros((1, L), jnp.float32)], axis=0)
    return sh * mask


def _cls_body(img, AwT, Ah, sx0, sx1, sx2, s2x0, s2x1, s2x2, m1, m2,
              W1r, b1c, w2x0, w2x1, w2x2, b2c, Wfc, bfcc, L0, L1, L2,
              out_ref):
    dot = functools.partial(jnp.dot, preferred_element_type=jnp.float32)
    Rs = []
    for c in range(3):
        Bc = dot(img[c * _H:(c + 1) * _H, :], AwT[...])
        Rs.append(dot(Ah[...], Bc))
    Rst = jnp.concatenate(Rs, axis=0)
    sxs = (sx0[...], sx1[...], sx2[...])
    CXs = [dot(Rst, sxs[dx]) for dx in range(3)]
    mask1 = m1[...]
    pieces = []
    for dy in range(3):
        for dx in range(3):
            A = _rowsel(CXs[dx], dy, mask1)
            pieces.append(A.reshape(3, 128 * 128))
    U1 = jnp.concatenate(pieces, axis=0)
    Hm = _lrelu(dot(W1r[...], U1) + b1c[...])
    Hst = Hm.reshape(2048, 128)
    s2xs = (s2x0[...], s2x1[...], s2x2[...])
    w2xs = (w2x0[...], w2x1[...], w2x2[...])
    mask2 = m2[...]
    u2rows = []
    for dy in range(3):
        A2y = _rowsel(Hst, dy, mask2)
        u2rows.append(A2y.reshape(16, 64 * 128))
    U2y = jnp.concatenate(u2rows, axis=0)
    acc = None
    for dx in range(3):
        G = dot(w2xs[dx], U2y)
        Cdx = dot(G.reshape(2048, 128), s2xs[dx])
        acc = Cdx if acc is None else acc + Cdx
    H2 = _lrelu(acc.reshape(32, 64, 64) + b2c[...].reshape(32, 1, 1))
    hm = jnp.mean(H2, axis=(1, 2), keepdims=True).reshape(32, 1)
    pred = dot(Wfc[...], hm) + bfcc[...]
    out_ref[...] = (pred[0:1, 0:1] * L0[...] + pred[1:2, 0:1] * L1[...]
                    + pred[2:3, 0:1] * L2[...])


_cls_call = pl.pallas_call(
    _cls_body,
    out_shape=jax.ShapeDtypeStruct((_LUT_PAD // 128, 128), jnp.float32),
)


def _lut2d(L):
    return jnp.pad(L.reshape(-1), (0, _LUT_PAD - 3 * _D3)).reshape(
        _LUT_PAD // 128, 128)


def kernel(image_input, W1, b1, W2, b2, Wfc, bfc, LUT0, LUT1, LUT2):
    (Ah, AwT, sx0, sx1, sx2, s2x0, s2x1, s2x2, m1, m2) = _consts()
    img2d = image_input.reshape(3 * _H, _W)
    W1r = jnp.transpose(W1, (0, 2, 3, 1)).reshape(16, 27)
    W2t = jnp.transpose(W2, (3, 0, 2, 1))
    w2x = [W2t[dx].reshape(32, 48) for dx in range(3)]
    lut2d = _cls_call(
        img2d, AwT, Ah, sx0, sx1, sx2, s2x0, s2x1, s2x2, m1, m2,
        W1r, b1.reshape(16, 1), w2x[0], w2x[1], w2x[2], b2.reshape(32, 1),
        Wfc, bfc.reshape(3, 1), _lut2d(LUT0), _lut2d(LUT1), _lut2d(LUT2))
    out = _build_trilerp()(lut2d.reshape(_LUT_PAD),
                           image_input.reshape(3 * _NPIX))
    return out.reshape(1, 3, _H, _W)


# speedup vs baseline: 998.1195x; 1.4837x over previous
import functools

import jax
import jax.numpy as jnp
from jax import lax
from jax.experimental import pallas as pl
from jax.experimental.pallas import tpu as pltpu
from jax.experimental.pallas import tpu_sc as plsc

_DIM = 33
_D2 = _DIM * _DIM
_D3 = _DIM * _D2
_H, _W = 1080, 1920
_NPIX = _H * _W
_NWORKER = 32
_PPW = _NPIX // _NWORKER
_CHUNK = 1200
_NCHUNK = _PPW // _CHUNK
_GROUPS = _CHUNK // 16
_LUT_PAD = 108544

@functools.lru_cache(maxsize=1)
def _build_trilerp():
    mesh = plsc.VectorSubcoreMesh(core_axis_name="c", subcore_axis_name="s")
    return functools.partial(
        pl.kernel,
        mesh=mesh,
        compiler_params=pltpu.CompilerParams(needs_layout_passes=False),
        out_type=jax.ShapeDtypeStruct((3 * _NPIX,), jnp.float32),
        scratch_types=[
            pltpu.VMEM((_LUT_PAD,), jnp.float32),
        ] + [pltpu.VMEM((_CHUNK,), jnp.float32) for _ in range(12)] + [
            pltpu.SemaphoreType.DMA,
            pltpu.SemaphoreType.DMA,
            pltpu.SemaphoreType.DMA,
            pltpu.SemaphoreType.DMA,
        ],
    )(_trilerp_body)


def _trilerp_body(lut_hbm, img_hbm, out_hbm, lut_v,
                  ir0, ig0, ib0, ir1, ig1, ib1,
                  or0, og0, ob0, or1, og1, ob1,
                  sin0, sin1, sout0, sout1):
    in_bufs = ((ir0, ig0, ib0), (ir1, ig1, ib1))
    out_bufs = ((or0, og0, ob0), (or1, og1, ob1))
    sem_in = (sin0, sin1)
    sem_out = (sout0, sout1)
    wid = lax.axis_index("s") * 2 + lax.axis_index("c")
    base = wid * _PPW
    pltpu.sync_copy(lut_hbm, lut_v)

    def start_in(j, b):
        off = base + j * _CHUNK
        for c in range(3):
            pltpu.async_copy(img_hbm.at[pl.ds(c * _NPIX + off, _CHUNK)],
                             in_bufs[b][c], sem_in[b])

    def wait_in(b):
        for c in range(3):
            pltpu.make_async_copy(img_hbm.at[pl.ds(0, _CHUNK)],
                                  in_bufs[b][c], sem_in[b]).wait()

    def start_out(j, b):
        off = base + j * _CHUNK
        for c in range(3):
            pltpu.async_copy(out_bufs[b][c],
                             out_hbm.at[pl.ds(c * _NPIX + off, _CHUNK)],
                             sem_out[b])

    def wait_out(b):
        for c in range(3):
            pltpu.make_async_copy(
                out_bufs[b][c],
                out_hbm.at[pl.ds(0, _CHUNK)], sem_out[b]).wait()

    def compute_chunk(in_v, out_v):
        @plsc.parallel_loop(0, _GROUPS, 1, unroll=4)
        def grp(i):
            s = i * 16
            r = in_v[0][pl.ds(s, 16)] * float(_DIM - 1)
            g = in_v[1][pl.ds(s, 16)] * float(_DIM - 1)
            b = in_v[2][pl.ds(s, 16)] * float(_DIM - 1)
            r0 = jnp.clip(r.astype(jnp.int32), 0, _DIM - 2)
            g0 = jnp.clip(g.astype(jnp.int32), 0, _DIM - 2)
            b0 = jnp.clip(b.astype(jnp.int32), 0, _DIM - 2)
            fr = r - r0.astype(jnp.float32)
            fg = g - g0.astype(jnp.float32)
            fb = b - b0.astype(jnp.float32)
            idx = r0 * _D2 + g0 * _DIM + b0
            for c in range(3):
                p = idx + c * _D3
                v000 = plsc.load_gather(lut_v, [p])
                v001 = plsc.load_gather(lut_v, [p + 1])
                v010 = plsc.load_gather(lut_v, [p + _DIM])
                v011 = plsc.load_gather(lut_v, [p + (_DIM + 1)])
                v100 = plsc.load_gather(lut_v, [p + _D2])
                v101 = plsc.load_gather(lut_v, [p + (_D2 + 1)])
                v110 = plsc.load_gather(lut_v, [p + (_D2 + _DIM)])
                v111 = plsc.load_gather(lut_v, [p + (_D2 + _DIM + 1)])
                c00 = v000 + fb * (v001 - v000)
                c01 = v010 + fb * (v011 - v010)
                c10 = v100 + fb * (v101 - v100)
                c11 = v110 + fb * (v111 - v110)
                c0 = c00 + fg * (c01 - c00)
                c1 = c10 + fg * (c11 - c10)
                out_v[c][pl.ds(s, 16)] = c0 + fr * (c1 - c0)

    start_in(0, 0)
    start_in(1, 1)

    def chunk_body(jj, carry):
        for b in range(2):
            j = jj * 2 + b
            wait_in(b)

            @pl.when(jj >= 1)
            def _():
                wait_out(b)

            compute_chunk(in_bufs[b], out_bufs[b])
            start_out(j, b)

            @pl.when(jj < _NCHUNK // 2 - 1)
            def _():
                start_in(j + 2, b)
        return carry

    lax.fori_loop(0, _NCHUNK // 2, chunk_body, 0)
    wait_out(0)
    wait_out(1)


@functools.lru_cache(maxsize=1)
def _consts():
    import numpy as np

    def wmat(in_size, out_size):
        inv_scale = np.float32(1.0) / np.float32(out_size / in_size)
        kernel_scale = max(float(inv_scale), 1.0)
        sample_f = ((np.arange(out_size, dtype=np.float32) + 0.5) * inv_scale
                    - 0.5)
        x = np.abs(sample_f[None, :]
                   - np.arange(in_size, dtype=np.float32)[:, None])
        x = x / np.float32(kernel_scale)
        w = np.maximum(np.float32(0), np.float32(1) - x).astype(np.float32)
        total = w.sum(axis=0, keepdims=True, dtype=np.float32)
        w = np.where(np.abs(total) > 1000.0 * np.finfo(np.float32).eps,
                     w / np.where(total != 0, total, 1), 0).astype(np.float32)
        valid = (sample_f >= -0.5) & (sample_f <= in_size - 0.5)
        return np.where(valid[None, :], w, 0).astype(np.float32)

    Ah = np.ascontiguousarray(wmat(_H, 256).T)
    AwT = wmat(_W, 256)
    sxT = []
    for dx in range(3):
        m = np.zeros((256, 128), np.float32)
        for x in range(128):
            j = 2 * x + dx
            if j < 256:
                m[j, x] = 1.0
        sxT.append(m)
    s2xT = []
    for dx in range(3):
        m = np.zeros((128, 64), np.float32)
        for x in range(64):
            j = 2 * x + dx
            if j < 128:
                m[j, x] = 1.0
        s2xT.append(m)
    m1 = np.ones((384, 1), np.float32)
    for c in range(3):
        m1[c * 128 + 127, 0] = 0.0
    m2 = np.ones((1024, 1), np.float32)
    for c in range(16):
        m2[c * 64 + 63, 0] = 0.0
    return (Ah, AwT, sxT[0], sxT[1], sxT[2], s2xT[0], s2xT[1], s2xT[2],
            m1, m2)


def _lrelu(x):
    return jnp.where(x > 0, x, 0.2 * x)


def _rowsel(M, dy, mask):
    n, L = M.shape
    M3 = M.reshape(n // 2, 2, L)
    even = M3[:, 0, :].reshape(n // 2, L)
    if dy == 0:
        return even
    if dy == 1:
        return M3[:, 1, :].reshape(n // 2, L)
    sh = jnp.concatenate([even[1:], jnp.zeros((1, L), jnp.float32)], axis=0)
    return sh * mask


def _cls_body(img, AwT, Ah, sx0, sx1, sx2, s2x0, s2x1, s2x2, m1, m2,
              W1r, b1c, w2x0, w2x1, w2x2, b2c, Wfc, bfcc, L0, L1, L2,
              out_ref):
    dot = functools.partial(jnp.dot, preferred_element_type=jnp.float32)
    Rs = []
    for c in range(3):
        Bc = dot(img[c * _H:(c + 1) * _H, :], AwT[...])
        Rs.append(dot(Ah[...], Bc))
    Rst = jnp.concatenate(Rs, axis=0)
    sxs = (sx0[...], sx1[...], sx2[...])
    CXs = [dot(Rst, sxs[dx]) for dx in range(3)]
    mask1 = m1[...]
    pieces = []
    for dy in range(3):
        for dx in range(3):
            A = _rowsel(CXs[dx], dy, mask1)
            pieces.append(A.reshape(3, 128 * 128))
    U1 = jnp.concatenate(pieces, axis=0)
    Hm = _lrelu(dot(W1r[...], U1) + b1c[...])
    Hst = Hm.reshape(2048, 128)
    s2xs = (s2x0[...], s2x1[...], s2x2[...])
    w2xs = (w2x0[...], w2x1[...], w2x2[...])
    mask2 = m2[...]
    u2rows = []
    for dy in range(3):
        A2y = _rowsel(Hst, dy, mask2)
        u2rows.append(A2y.reshape(16, 64 * 128))
    U2y = jnp.concatenate(u2rows, axis=0)
    acc = None
    for dx in range(3):
        G = dot(w2xs[dx], U2y)
        Cdx = dot(G.reshape(2048, 128), s2xs[dx])
        acc = Cdx if acc is None else acc + Cdx
    H2 = _lrelu(acc.reshape(32, 64, 64) + b2c[...].reshape(32, 1, 1))
    hm = jnp.mean(H2, axis=(1, 2), keepdims=True).reshape(32, 1)
    pred = dot(Wfc[...], hm) + bfcc[...]
    out_ref[...] = (pred[0:1, 0:1] * L0[...] + pred[1:2, 0:1] * L1[...]
                    + pred[2:3, 0:1] * L2[...])


_cls_call = pl.pallas_call(
    _cls_body,
    out_shape=jax.ShapeDtypeStruct((_LUT_PAD // 128, 128), jnp.float32),
)


def _lut2d(L):
    return jnp.pad(L.reshape(-1), (0, _LUT_PAD - 3 * _D3)).reshape(
        _LUT_PAD // 128, 128)


def kernel(image_input, W1, b1, W2, b2, Wfc, bfc, LUT0, LUT1, LUT2):
    (Ah, AwT, sx0, sx1, sx2, s2x0, s2x1, s2x2, m1, m2) = _consts()
    img2d = image_input.reshape(3 * _H, _W)
    W1r = jnp.transpose(W1, (0, 2, 3, 1)).reshape(16, 27)
    W2t = jnp.transpose(W2, (3, 0, 2, 1))
    w2x = [W2t[dx].reshape(32, 48) for dx in range(3)]
    lut2d = _cls_call(
        img2d, AwT, Ah, sx0, sx1, sx2, s2x0, s2x1, s2x2, m1, m2,
        W1r, b1.reshape(16, 1), w2x[0], w2x[1], w2x[2], b2.reshape(32, 1),
        Wfc, bfc.reshape(3, 1), _lut2d(LUT0), _lut2d(LUT1), _lut2d(LUT2))
    out = _build_trilerp()(lut2d.reshape(_LUT_PAD),
                           image_input.reshape(3 * _NPIX))
    return out.reshape(1, 3, _H, _W)
